# Initial kernel scaffold; baseline (speedup 1.0000x reference)
import jax
import jax.numpy as jnp
from jax.experimental import pallas as pl


def kernel(prob, z_i, z_j):
    raise NotImplementedError("write your pallas kernel here")


# speedup vs baseline: 4.5901x; 4.5901x over previous
import functools

import jax
import jax.numpy as jnp
from jax import lax
from jax.experimental import pallas as pl
from jax.experimental.pallas import tpu as pltpu
from jax.experimental.pallas import tpu_sc as plsc

_TEMPERATURE = 0.5
_K = 64
_NC = 10
_BATCH = 16384
_DIM = 128
_ROWS = 2 * _K * _NC
_BLK = 256


def _topk_body(prob_ref, out_ref, cur_ref):
    cur_ref[...] = prob_ref[...]
    col = lax.broadcasted_iota(jnp.int32, (16, _BATCH), 1)
    col64 = lax.broadcasted_iota(jnp.int32, (16, _K), 1)

    def step(i, idxacc):
        cur = cur_ref[...]
        m = jnp.max(cur, axis=1, keepdims=True)
        cand = jnp.where(cur == m, col, jnp.int32(2**30))
        sel = jnp.min(cand, axis=1, keepdims=True)
        cur_ref[...] = jnp.where(col == sel, -jnp.inf, cur)
        return jnp.where(col64 == i, sel, idxacc)

    out_ref[...] = lax.fori_loop(0, _K, step, jnp.zeros((16, _K), jnp.int32))


def _topk_tc(prob16):
    return pl.pallas_call(
        _topk_body,
        out_shape=jax.ShapeDtypeStruct((16, _K), jnp.int32),
        scratch_shapes=[pltpu.VMEM((16, _BATCH), jnp.float32)],
    )(prob16)


def _gather_sc_body(idx_hbm, zi_hbm, zj_hbm, out_hbm, idx_v, rows_v, sem):
    ci = lax.axis_index("c")
    s = lax.axis_index("s")

    @pl.when(s < _NC)
    def _():
        pltpu.sync_copy(idx_hbm.at[s], idx_v)

        @pl.when(ci == 0)
        def _():
            pltpu.async_copy(zi_hbm.at[idx_v], rows_v, sem).wait()
            pltpu.sync_copy(rows_v, out_hbm.at[pl.ds(s * 2 * _K, _K)])

        @pl.when(ci == 1)
        def _():
            pltpu.async_copy(zj_hbm.at[idx_v], rows_v, sem).wait()
            pltpu.sync_copy(rows_v, out_hbm.at[pl.ds(s * 2 * _K + _K, _K)])


def _gather_sc(topk_idx, z_i, z_j):
    mesh = plsc.VectorSubcoreMesh(core_axis_name="c", subcore_axis_name="s")
    k = functools.partial(
        pl.kernel,
        mesh=mesh,
        out_type=jax.ShapeDtypeStruct((_ROWS, _DIM), jnp.float32),
        scratch_types=[
            pltpu.VMEM((_K,), jnp.int32),
            pltpu.VMEM((_K, _DIM), jnp.float32),
            pltpu.SemaphoreType.DMA,
        ],
    )(_gather_sc_body)
    return k(topk_idx, z_i, z_j)


def _loss_body(u_blk_ref, u_all_ref, out_ref):
    i = pl.program_id(0)
    u_blk = u_blk_ref[...]
    u_all = u_all_ref[...]

    n2_all = jnp.sum(u_all * u_all, axis=1, keepdims=True)
    na_all = jnp.sqrt(n2_all)
    n2_blk = jnp.sum(u_blk * u_blk, axis=1, keepdims=True)
    na_blk = jnp.sqrt(n2_blk)

    dots = lax.dot_general(
        u_blk, u_all, (((1,), (1,)), ((), ())),
        preferred_element_type=jnp.float32,
        precision=lax.Precision.HIGHEST,
    )
    denom = jnp.maximum(na_blk * na_all.T, 1e-8)
    e = jnp.exp(dots / denom / _TEMPERATURE)

    gi = lax.broadcasted_iota(jnp.int32, (_BLK, _ROWS), 0) + i * _BLK
    gj = lax.broadcasted_iota(jnp.int32, (_BLK, _ROWS), 1)
    own = (gi // (2 * _K)) == (gj // (2 * _K))
    posm = own & ((gj % (2 * _K)) < _K)

    total = jnp.sum(e, axis=1)
    own_sum = jnp.sum(jnp.where(own, e, 0.0), axis=1)
    pos_sum = jnp.sum(jnp.where(posm, e, 0.0), axis=1)
    part = jnp.sum(jnp.log(total - own_sum) - jnp.log(pos_sum))

    @pl.when(i == 0)
    def _():
        out_ref[0, 0] = 0.0

    out_ref[0, 0] += part


def _loss_tc(u):
    out = pl.pallas_call(
        _loss_body,
        grid=(_ROWS // _BLK,),
        in_specs=[
            pl.BlockSpec((_BLK, _DIM), lambda i: (i, 0)),
            pl.BlockSpec((_ROWS, _DIM), lambda i: (0, 0)),
        ],
        out_specs=pl.BlockSpec(memory_space=pltpu.SMEM),
        out_shape=jax.ShapeDtypeStruct((1, 1), jnp.float32),
    )(u, u)
    return out[0, 0] / _ROWS


def kernel(prob, z_i, z_j):
    prob16 = jnp.concatenate(
        [prob.T, jnp.full((16 - _NC, _BATCH), -jnp.inf, jnp.float32)], axis=0
    )
    topk_idx = _topk_tc(prob16)
    u = _gather_sc(topk_idx, z_i, z_j)
    return _loss_tc(u)
